# baseline (speedup 1.0000x reference)
import dataclasses
import functools

import jax
import jax.numpy as jnp
from jax import lax
from jax.experimental import pallas as pl
from jax.experimental.pallas import tpu as pltpu
from jax.experimental.pallas import tpu_sc as plsc

N = 10000
E = 320000
D = 128
NC = 2
NS = 16
L = 16
NW = NC * NS
CH = 128
NCHUNK = 79
EPTP = NCHUNK * CH
EPAD = NW * EPTP
NRC = N // 80

def _sc_round(hb, src, dst3, ew):
    mesh = plsc.VectorSubcoreMesh(core_axis_name="c", subcore_axis_name="s")
    cp = pltpu.CompilerParams()
    if "needs_layout_passes" in pltpu.CompilerParams.__dataclass_fields__:
        cp = dataclasses.replace(cp, needs_layout_passes=False)

    @functools.partial(
        pl.kernel,
        compiler_params=cp,
        out_type=jax.ShapeDtypeStruct((NC, N, D), jnp.float32),
        mesh=mesh,
        scratch_types=[
            pltpu.VMEM_SHARED((N, D), jnp.float32),
            pltpu.VMEM((CH, D), jnp.float32),
            pltpu.VMEM((CH, D), jnp.float32),
            pltpu.VMEM((NCHUNK, CH), jnp.int32),
            pltpu.VMEM((CH,), jnp.int32),
            pltpu.VMEM((CH,), jnp.int32),
            pltpu.VMEM((CH,), jnp.float32),
            pltpu.VMEM((CH,), jnp.float32),
            pltpu.SemaphoreType.DMA,
            pltpu.SemaphoreType.DMA,
            pltpu.SemaphoreType.DMA,
            pltpu.SemaphoreType.DMA,
        ],
    )
    def k(h_hbm, src_hbm, dst_hbm, ew_hbm, out_hbm,
          acc, rb0, rb1, didx, sb0, sb1, wb0, wb1,
          gsem0, gsem1, isem0, isem1):
        c = lax.axis_index("c")
        s = lax.axis_index("s")
        wid = c * NS + s

        pltpu.sync_copy(dst_hbm.at[wid], didx)

        @pl.loop(0, 80)
        def _zero_fill(r):
            for g in range(D // L):
                rb0[r, pl.ds(g * L, L)] = jnp.zeros((L,), jnp.float32)

        for i in range((NRC + NS - 1) // NS):
            j = i * NS + s

            @pl.when(j < NRC)
            def _():
                pltpu.sync_copy(rb0.at[pl.ds(0, 80)], acc.at[pl.ds(j * 80, 80)])

        plsc.subcore_barrier()

        def idx_start(ci, sb, wb, sem):
            base = wid * EPTP + ci * CH
            pltpu.async_copy(src_hbm.at[pl.ds(base, CH)], sb, sem)
            pltpu.async_copy(ew_hbm.at[pl.ds(base, CH)], wb, sem)

        def idx_wait(ci, sb, wb, sem):
            base = wid * EPTP + ci * CH
            pltpu.make_async_copy(src_hbm.at[pl.ds(base, CH)], sb, sem).wait()
            pltpu.make_async_copy(ew_hbm.at[pl.ds(base, CH)], wb, sem).wait()

        def gather_start(sb, rb, sem):
            pltpu.async_copy(h_hbm.at[sb], rb, sem)

        def gather_wait(sb, rb, sem):
            pltpu.make_async_copy(h_hbm.at[sb], rb, sem).wait()

        def scale(rb, wb):
            @plsc.parallel_loop(0, CH, unroll=4)
            def _scale(r):
                w16 = plsc.load_gather(
                    wb, [jnp.broadcast_to(r, (L,)).astype(jnp.int32)]
                )
                for g in range(D // L):
                    rb[r, pl.ds(g * L, L)] = rb[r, pl.ds(g * L, L)] * w16

        def scatter_add(ci, rb):
            pltpu.sync_copy(rb, acc.at[didx.at[ci]], add=True)

        idx_start(0, sb0, wb0, isem0)
        idx_start(1, sb1, wb1, isem1)
        idx_wait(0, sb0, wb0, isem0)
        gather_start(sb0, rb0, gsem0)

        @pl.loop(0, NCHUNK - 1, step=2)
        def _pipe(ci):
            idx_wait(ci + 1, sb1, wb1, isem1)
            gather_start(sb1, rb1, gsem1)
            gather_wait(sb0, rb0, gsem0)
            scale(rb0, wb0)
            idx_start(ci + 2, sb0, wb0, isem0)
            scatter_add(ci, rb0)
            idx_wait(ci + 2, sb0, wb0, isem0)
            gather_start(sb0, rb0, gsem0)
            gather_wait(sb1, rb1, gsem1)
            scale(rb1, wb1)
            idx_start(ci + 3, sb1, wb1, isem1)
            scatter_add(ci + 1, rb1)

        gather_wait(sb0, rb0, gsem0)
        scale(rb0, wb0)
        scatter_add(NCHUNK - 1, rb0)
        idx_wait(NCHUNK, sb1, wb1, isem1)

        plsc.subcore_barrier()

        for i in range((NRC + NS - 1) // NS):
            j = i * NS + s

            @pl.when(j < NRC)
            def _():
                pltpu.sync_copy(
                    acc.at[pl.ds(j * 80, 80)],
                    out_hbm.at[c, pl.ds(j * 80, 80)],
                )

    return k(hb, src, dst3, ew)


_BLK = 2000


def _tc_combine(p):
    def body(p_ref, o_ref):
        o_ref[...] = p_ref[0] + p_ref[1]

    return pl.pallas_call(
        body,
        out_shape=jax.ShapeDtypeStruct((N, D), jnp.float32),
        grid=(N // _BLK,),
        in_specs=[pl.BlockSpec((NC, _BLK, D), lambda i: (0, i, 0))],
        out_specs=pl.BlockSpec((_BLK, D), lambda i: (i, 0)),
    )(p)


def _tc_final(x, h1, q, W, b2):
    def body(x_ref, h1_ref, q_ref, w_ref, b_ref, o_ref):
        sm = (x_ref[...] + h1_ref[...] + q_ref[0] + q_ref[1]) * (1.0 / 3.0)
        o_ref[...] = (
            lax.dot_general(
                sm,
                w_ref[...],
                (((1,), (1,)), ((), ())),
                precision=lax.Precision.HIGHEST,
            )
            + b_ref[...]
        )

    return pl.pallas_call(
        body,
        out_shape=jax.ShapeDtypeStruct((N, D), jnp.float32),
        grid=(N // _BLK,),
        in_specs=[
            pl.BlockSpec((_BLK, D), lambda i: (i, 0)),
            pl.BlockSpec((_BLK, D), lambda i: (i, 0)),
            pl.BlockSpec((NC, _BLK, D), lambda i: (0, i, 0)),
            pl.BlockSpec((D, D), lambda i: (0, 0)),
            pl.BlockSpec((1, D), lambda i: (0, 0)),
        ],
        out_specs=pl.BlockSpec((_BLK, D), lambda i: (i, 0)),
    )(x, h1, q, W, b2)


def kernel(x, edge_index, edge_weight, W, b):
    src = edge_index[0].astype(jnp.int32)
    dst = edge_index[1].astype(jnp.int32)
    ew = edge_weight.astype(jnp.float32)
    src_p = jnp.concatenate([src, jnp.zeros(EPAD - E + CH, jnp.int32)])
    ew_p = jnp.concatenate([ew, jnp.zeros(EPAD - E + CH, jnp.float32)])
    dst3 = jnp.concatenate([dst, jnp.zeros(EPAD - E, jnp.int32)]).reshape(
        NW, NCHUNK, CH
    )

    p = _sc_round(x, src_p, dst3, ew_p)
    h1 = _tc_combine(p)
    q = _sc_round(h1, src_p, dst3, ew_p)
    return _tc_final(x, h1, q, W, b.reshape(1, D))
